# speedup vs baseline: 1.9803x; 1.9803x over previous
import functools

import jax
import jax.numpy as jnp
from jax import lax
from jax.experimental import pallas as pl
from jax.experimental.pallas import tpu as pltpu
from jax.experimental.pallas import tpu_sc as plsc

D_MODEL = 1024
NUM_CORES = 2
NUM_SUBCORES = 16
NUM_WORKERS = NUM_CORES * NUM_SUBCORES
CHUNK = 32


@functools.partial(jax.jit, static_argnames=("b_per_w", "n_chunks"))
def _gather_rows(pe, idx, *, b_per_w, n_chunks):
    total = idx.shape[0]
    mesh = plsc.VectorSubcoreMesh(core_axis_name="c", subcore_axis_name="s")

    @functools.partial(
        pl.kernel,
        out_type=jax.ShapeDtypeStruct((total, D_MODEL), jnp.float32),
        mesh=mesh,
        scratch_types=[
            pltpu.VMEM((b_per_w,), jnp.int32),
            pltpu.VMEM((CHUNK, D_MODEL), jnp.float32),
            pltpu.SemaphoreType.DMA,
        ],
    )
    def body(pe_hbm, idx_hbm, out_hbm, idx_v, rows_v, sem):
        wid = lax.axis_index("s") * NUM_CORES + lax.axis_index("c")
        base = wid * b_per_w
        pltpu.sync_copy(idx_hbm.at[pl.ds(base, b_per_w)], idx_v)

        @pl.loop(0, n_chunks)
        def _chunk(i):
            off = i * CHUNK
            pltpu.async_copy(
                pe_hbm.at[idx_v.at[pl.ds(off, CHUNK)]], rows_v, sem
            ).wait()
            pltpu.sync_copy(rows_v, out_hbm.at[pl.ds(base + off, CHUNK)])

    return body(pe, idx)


def kernel(position_ids, pe):
    idx = position_ids.reshape(-1)
    total = idx.shape[0]
    b_per_w = total // NUM_WORKERS
    out = _gather_rows(pe, idx, b_per_w=b_per_w, n_chunks=b_per_w // CHUNK)
    return out.reshape(position_ids.shape + (pe.shape[1],))


# speedup vs baseline: 2.3830x; 1.2034x over previous
import functools

import jax
import jax.numpy as jnp
from jax import lax
from jax.experimental import pallas as pl
from jax.experimental.pallas import tpu as pltpu
from jax.experimental.pallas import tpu_sc as plsc

D_MODEL = 1024
NUM_CORES = 2
NUM_SUBCORES = 16
NUM_WORKERS = NUM_CORES * NUM_SUBCORES
CHUNK = 32


@functools.partial(jax.jit, static_argnames=("b_per_w", "n_chunks"))
def _gather_rows(pe, idx, *, b_per_w, n_chunks):
    total = idx.shape[0]
    mesh = plsc.VectorSubcoreMesh(core_axis_name="c", subcore_axis_name="s")

    @functools.partial(
        pl.kernel,
        out_type=jax.ShapeDtypeStruct((total, D_MODEL), jnp.float32),
        mesh=mesh,
        scratch_types=[
            pltpu.VMEM((b_per_w,), jnp.int32),
            pltpu.VMEM((2, CHUNK, D_MODEL), jnp.float32),
            pltpu.SemaphoreType.DMA,
            pltpu.SemaphoreType.DMA,
            pltpu.SemaphoreType.DMA,
            pltpu.SemaphoreType.DMA,
        ],
    )
    def body(pe_hbm, idx_hbm, out_hbm, idx_v, rows_v, g0, g1, s0, s1):
        sems_g = (g0, g1)
        sems_s = (s0, s1)
        wid = lax.axis_index("s") * NUM_CORES + lax.axis_index("c")
        base = wid * b_per_w
        pltpu.sync_copy(idx_hbm.at[pl.ds(base, b_per_w)], idx_v)

        def gather_start(chunk, b):
            pltpu.async_copy(
                pe_hbm.at[idx_v.at[pl.ds(chunk * CHUNK, CHUNK)]],
                rows_v.at[b],
                sems_g[b],
            )

        def gather_wait(b):
            pltpu.make_async_copy(
                pe_hbm.at[idx_v.at[pl.ds(0, CHUNK)]], rows_v.at[b], sems_g[b]
            ).wait()

        def scatter_start(chunk, b):
            pltpu.async_copy(
                rows_v.at[b],
                out_hbm.at[pl.ds(base + chunk * CHUNK, CHUNK)],
                sems_s[b],
            )

        def scatter_wait(b):
            pltpu.make_async_copy(
                rows_v.at[b], out_hbm.at[pl.ds(base, CHUNK)], sems_s[b]
            ).wait()

        gather_start(0, 0)

        @pl.loop(0, n_chunks, step=2)
        def _pair(i):
            for b in range(2):
                j = i + b
                nxt = j + 1

                @pl.when(nxt < n_chunks)
                def _():
                    @pl.when(nxt >= 2)
                    def _():
                        scatter_wait(1 - b)

                    gather_start(nxt, 1 - b)

                gather_wait(b)
                scatter_start(j, b)

        scatter_wait(0)
        scatter_wait(1)

    return body(pe, idx)


def kernel(position_ids, pe):
    idx = position_ids.reshape(-1)
    total = idx.shape[0]
    b_per_w = total // NUM_WORKERS
    out = _gather_rows(pe, idx, b_per_w=b_per_w, n_chunks=b_per_w // CHUNK)
    return out.reshape(position_ids.shape + (pe.shape[1],))
